# Initial kernel scaffold; baseline (speedup 1.0000x reference)
import jax
import jax.numpy as jnp
from jax.experimental import pallas as pl


def kernel(pd_scores, pd_bboxes, anc_points, gt_labels, gt_bboxes, mask_gt, stride):
    raise NotImplementedError("write your pallas kernel here")


# speedup vs baseline: 8.6554x; 8.6554x over previous
import functools

import jax
import jax.numpy as jnp
from jax.experimental import pallas as pl
from jax.experimental.pallas import tpu as pltpu

NUM_CLASSES = 80
CENTER_RADIUS = 1.5
NMAX = 64
BLK_A = 2048
INF = float("inf")


def _assign_body(gt_ref, lbl_ref, mgt_ref, ancx_ref, ancy_ref, stride_ref,
                 lab_out, bx1_out, by1_out, bx2_out, by2_out, sc_out,
                 fg_out, idx_out):
    xs = ancx_ref[0, :]
    ys = ancy_ref[0, :]
    radius = CENTER_RADIUS * stride_ref[0, 0, :]

    best = jnp.full((BLK_A,), INF, dtype=jnp.float32)
    bidx = jnp.zeros((BLK_A,), dtype=jnp.int32)
    blab = jnp.zeros((BLK_A,), dtype=jnp.int32)
    bx1 = jnp.zeros((BLK_A,), dtype=jnp.float32)
    by1 = jnp.zeros((BLK_A,), dtype=jnp.float32)
    bx2 = jnp.zeros((BLK_A,), dtype=jnp.float32)
    by2 = jnp.zeros((BLK_A,), dtype=jnp.float32)

    for g in range(NMAX):
        x1 = gt_ref[0, g, 0]
        y1 = gt_ref[0, g, 1]
        x2 = gt_ref[0, g, 2]
        y2 = gt_ref[0, g, 3]
        lblg = lbl_ref[0, g, 0]
        mg = mgt_ref[0, g, 0]
        gcx = (x1 + x2) * 0.5
        gcy = (y1 + y2) * 0.5
        area = (x2 - x1) * (y2 - y1)
        area_eff = jnp.where(mg > 0, area, INF)

        l = xs - x1
        t = ys - y1
        r = x2 - xs
        b = y2 - ys
        m = jnp.minimum(jnp.minimum(l, t), jnp.minimum(r, b))
        cm = jnp.minimum(radius - jnp.abs(xs - gcx), radius - jnp.abs(ys - gcy))
        m = jnp.minimum(m, cm)
        cand = jnp.where(m > 0, area_eff, INF)
        upd = cand < best
        best = jnp.where(upd, cand, best)
        bidx = jnp.where(upd, g, bidx)
        blab = jnp.where(upd, lblg, blab)
        bx1 = jnp.where(upd, x1, bx1)
        by1 = jnp.where(upd, y1, by1)
        bx2 = jnp.where(upd, x2, bx2)
        by2 = jnp.where(upd, y2, by2)

    fg = best < INF
    lab = jnp.where(fg, blab, NUM_CLASSES)
    lab_out[0, 0, :] = lab
    fg_out[0, 0, :] = fg.astype(jnp.int32)
    idx_out[0, 0, :] = bidx
    bx1_out[0, 0, :] = jnp.where(fg, bx1, 0.0)
    by1_out[0, 0, :] = jnp.where(fg, by1, 0.0)
    bx2_out[0, 0, :] = jnp.where(fg, bx2, 0.0)
    by2_out[0, 0, :] = jnp.where(fg, by2, 0.0)
    cls = jax.lax.broadcasted_iota(jnp.int32, (BLK_A, NUM_CLASSES), 1)
    sc_out[0, :, :] = (cls == lab[:, None]).astype(jnp.float32)


def kernel(pd_scores, pd_bboxes, anc_points, gt_labels, gt_bboxes, mask_gt, stride):
    bs, na = stride.shape[0], stride.shape[1]
    nap = ((na + BLK_A - 1) // BLK_A) * BLK_A
    pad = nap - na

    anc_x = jnp.pad(anc_points[:, 0], (0, pad)).reshape(1, nap)
    anc_y = jnp.pad(anc_points[:, 1], (0, pad)).reshape(1, nap)
    stride3d = jnp.pad(stride[:, :, 0], ((0, 0), (0, pad))).reshape(bs, 1, nap)
    gt_lab = gt_labels.astype(jnp.int32)
    mgt = mask_gt

    n_blk = nap // BLK_A
    grid = (bs, n_blk)

    out_shapes = (
        jax.ShapeDtypeStruct((bs, 1, nap), jnp.int32),
        jax.ShapeDtypeStruct((bs, 1, nap), jnp.float32),
        jax.ShapeDtypeStruct((bs, 1, nap), jnp.float32),
        jax.ShapeDtypeStruct((bs, 1, nap), jnp.float32),
        jax.ShapeDtypeStruct((bs, 1, nap), jnp.float32),
        jax.ShapeDtypeStruct((bs, nap, NUM_CLASSES), jnp.float32),
        jax.ShapeDtypeStruct((bs, 1, nap), jnp.int32),
        jax.ShapeDtypeStruct((bs, 1, nap), jnp.int32),
    )

    smem = functools.partial(pl.BlockSpec, memory_space=pltpu.SMEM)
    anc_in = pl.BlockSpec((1, BLK_A), lambda b, j: (0, j))
    vec_in = pl.BlockSpec((1, 1, BLK_A), lambda b, j: (b, 0, j))
    vec_out = pl.BlockSpec((1, 1, BLK_A), lambda b, j: (b, 0, j))
    sc_spec = pl.BlockSpec((1, BLK_A, NUM_CLASSES), lambda b, j: (b, j, 0))

    outs = pl.pallas_call(
        _assign_body,
        grid=grid,
        in_specs=[
            smem((1, NMAX, 4), lambda b, j: (b, 0, 0)),
            smem((1, NMAX, 1), lambda b, j: (b, 0, 0)),
            smem((1, NMAX, 1), lambda b, j: (b, 0, 0)),
            anc_in,
            anc_in,
            vec_in,
        ],
        out_specs=(vec_out, vec_out, vec_out, vec_out, vec_out, sc_spec,
                   vec_out, vec_out),
        out_shape=out_shapes,
    )(gt_bboxes, gt_lab, mgt, anc_x, anc_y, stride3d)

    lab, x1o, y1o, x2o, y2o, sc, fg, gidx = outs
    target_labels = lab[:, 0, :na]
    target_bboxes = jnp.stack(
        [x1o[:, 0, :na], y1o[:, 0, :na], x2o[:, 0, :na], y2o[:, 0, :na]],
        axis=-1)
    target_scores = sc[:, :na, :]
    fg_mask = fg[:, 0, :na].astype(jnp.bool_)
    target_gt_idx = gidx[:, 0, :na]
    return (target_labels, target_bboxes, target_scores, fg_mask, target_gt_idx)


# speedup vs baseline: 11.8873x; 1.3734x over previous
import functools

import jax
import jax.numpy as jnp
from jax.experimental import pallas as pl
from jax.experimental.pallas import tpu as pltpu

NUM_CLASSES = 80
CENTER_RADIUS = 1.5
NMAX = 64
BLK_A = 2048
INF = float("inf")


def _assign_body(gt_ref, lbl_ref, mgt_ref, ancx_ref, ancy_ref, stride_ref,
                 lab_out, bx1_out, by1_out, bx2_out, by2_out, sc_out,
                 fg_out, idx_out):
    xs = ancx_ref[0, :]
    ys = ancy_ref[0, :]
    radius = CENTER_RADIUS * stride_ref[0, 0, :]

    best = jnp.full((BLK_A,), INF, dtype=jnp.float32)
    bidx = jnp.zeros((BLK_A,), dtype=jnp.int32)
    blab = jnp.zeros((BLK_A,), dtype=jnp.int32)
    bx1 = jnp.zeros((BLK_A,), dtype=jnp.float32)
    by1 = jnp.zeros((BLK_A,), dtype=jnp.float32)
    bx2 = jnp.zeros((BLK_A,), dtype=jnp.float32)
    by2 = jnp.zeros((BLK_A,), dtype=jnp.float32)

    for g in range(NMAX):
        x1 = gt_ref[0, g, 0]
        y1 = gt_ref[0, g, 1]
        x2 = gt_ref[0, g, 2]
        y2 = gt_ref[0, g, 3]
        lblg = lbl_ref[0, g, 0]
        mg = mgt_ref[0, g, 0]
        gcx = (x1 + x2) * 0.5
        gcy = (y1 + y2) * 0.5
        area = (x2 - x1) * (y2 - y1)
        area_eff = jnp.where(mg > 0, area, INF)

        l = xs - x1
        t = ys - y1
        r = x2 - xs
        b = y2 - ys
        m = jnp.minimum(jnp.minimum(l, t), jnp.minimum(r, b))
        cm = jnp.minimum(radius - jnp.abs(xs - gcx), radius - jnp.abs(ys - gcy))
        m = jnp.minimum(m, cm)
        cand = jnp.where(m > 0, area_eff, INF)
        upd = cand < best
        best = jnp.where(upd, cand, best)
        bidx = jnp.where(upd, g, bidx)
        blab = jnp.where(upd, lblg, blab)
        bx1 = jnp.where(upd, x1, bx1)
        by1 = jnp.where(upd, y1, by1)
        bx2 = jnp.where(upd, x2, bx2)
        by2 = jnp.where(upd, y2, by2)

    fg = best < INF
    lab = jnp.where(fg, blab, NUM_CLASSES)
    lab_out[0, 0, :] = lab
    fg_out[0, 0, :] = fg.astype(jnp.int32)
    idx_out[0, 0, :] = bidx
    bx1_out[0, 0, :] = jnp.where(fg, bx1, 0.0)
    by1_out[0, 0, :] = jnp.where(fg, by1, 0.0)
    bx2_out[0, 0, :] = jnp.where(fg, bx2, 0.0)
    by2_out[0, 0, :] = jnp.where(fg, by2, 0.0)
    cls = jax.lax.broadcasted_iota(jnp.int32, (BLK_A, NUM_CLASSES), 1)
    sc_out[0, :, :] = (cls == lab[:, None]).astype(jnp.float32)


def kernel(pd_scores, pd_bboxes, anc_points, gt_labels, gt_bboxes, mask_gt, stride):
    bs, na = stride.shape[0], stride.shape[1]

    anc_x = anc_points[:, 0].reshape(1, na)
    anc_y = anc_points[:, 1].reshape(1, na)
    stride3d = stride[:, :, 0].reshape(bs, 1, na)
    gt_lab = gt_labels.astype(jnp.int32)
    mgt = mask_gt

    n_blk = (na + BLK_A - 1) // BLK_A
    grid = (bs, n_blk)

    out_shapes = (
        jax.ShapeDtypeStruct((bs, 1, na), jnp.int32),
        jax.ShapeDtypeStruct((bs, 1, na), jnp.float32),
        jax.ShapeDtypeStruct((bs, 1, na), jnp.float32),
        jax.ShapeDtypeStruct((bs, 1, na), jnp.float32),
        jax.ShapeDtypeStruct((bs, 1, na), jnp.float32),
        jax.ShapeDtypeStruct((bs, na, NUM_CLASSES), jnp.float32),
        jax.ShapeDtypeStruct((bs, 1, na), jnp.int32),
        jax.ShapeDtypeStruct((bs, 1, na), jnp.int32),
    )

    smem = functools.partial(pl.BlockSpec, memory_space=pltpu.SMEM)
    anc_in = pl.BlockSpec((1, BLK_A), lambda b, j: (0, j))
    vec_in = pl.BlockSpec((1, 1, BLK_A), lambda b, j: (b, 0, j))
    vec_out = pl.BlockSpec((1, 1, BLK_A), lambda b, j: (b, 0, j))
    sc_spec = pl.BlockSpec((1, BLK_A, NUM_CLASSES), lambda b, j: (b, j, 0))

    outs = pl.pallas_call(
        _assign_body,
        grid=grid,
        in_specs=[
            smem((1, NMAX, 4), lambda b, j: (b, 0, 0)),
            smem((1, NMAX, 1), lambda b, j: (b, 0, 0)),
            smem((1, NMAX, 1), lambda b, j: (b, 0, 0)),
            anc_in,
            anc_in,
            vec_in,
        ],
        out_specs=(vec_out, vec_out, vec_out, vec_out, vec_out, sc_spec,
                   vec_out, vec_out),
        out_shape=out_shapes,
    )(gt_bboxes, gt_lab, mgt, anc_x, anc_y, stride3d)

    lab, x1o, y1o, x2o, y2o, sc, fg, gidx = outs
    target_labels = lab[:, 0, :]
    target_bboxes = jnp.stack(
        [x1o[:, 0, :], y1o[:, 0, :], x2o[:, 0, :], y2o[:, 0, :]], axis=-1)
    target_scores = sc
    fg_mask = fg[:, 0, :].astype(jnp.bool_)
    target_gt_idx = gidx[:, 0, :]
    return (target_labels, target_bboxes, target_scores, fg_mask, target_gt_idx)
